# Initial kernel scaffold; baseline (speedup 1.0000x reference)
import jax
import jax.numpy as jnp
from jax.experimental import pallas as pl


def kernel(in_list, o, i, W1, b1, W2, b2, W3, b3, W4, b4, W5, b5):
    raise NotImplementedError("write your pallas kernel here")


# speedup vs baseline: 1.2853x; 1.2853x over previous
import functools

import jax
import jax.numpy as jnp
from jax import lax
from jax.experimental import pallas as pl
from jax.experimental.pallas import tpu as pltpu

E = 8
IN_DIM = 1024
DIM = 2048
B = 4096
OUT_DIM = 3072

BM = 256
G = B // BM + E
P = G * BM


def _routing(idx):
    idx = idx.astype(jnp.int32)
    order = jnp.argsort(idx, stable=True).astype(jnp.int32)
    counts = jnp.bincount(idx, length=E)
    blocks = (counts + BM - 1) // BM
    ends_blk = jnp.cumsum(blocks)
    start_al = ((ends_blk - blocks) * BM).astype(jnp.int32)
    cum = (jnp.cumsum(counts) - counts).astype(jnp.int32)
    es = idx[order]
    rank = jnp.arange(B, dtype=jnp.int32) - cum[es]
    dst = start_al[es] + rank
    gidx = jnp.zeros((P,), jnp.int32).at[dst].set(order)
    pos = jnp.zeros((B,), jnp.int32).at[order].set(dst)
    be = jnp.searchsorted(ends_blk, jnp.arange(G), side="right")
    be = jnp.minimum(be, E - 1).astype(jnp.int32)
    return gidx, pos, be


def _gather_rows(table, idx):
    return jnp.take(table, idx, axis=0)


def _grouped_kernel(act, be_ref, x_ref, w_ref, b_ref, o_ref):
    y = lax.dot_general(
        x_ref[...], w_ref[0],
        (((1,), (1,)), ((), ())),
        preferred_element_type=jnp.float32,
    )
    y = y + b_ref[0]
    if act == "relu":
        y = jnp.maximum(y, 0.0)
    o_ref[...] = y


def _grouped_matmul(x, w, b, be, act, nsplit=1):
    rows = x.shape[0]
    _, n, k = w.shape
    bn = n // nsplit
    b2 = b.reshape(E, 1, n)
    grid_spec = pltpu.PrefetchScalarGridSpec(
        num_scalar_prefetch=1,
        grid=(rows // BM, nsplit),
        in_specs=[
            pl.BlockSpec((BM, k), lambda g, j, be: (g, 0)),
            pl.BlockSpec((1, bn, k), lambda g, j, be: (be[g], j, 0)),
            pl.BlockSpec((1, 1, bn), lambda g, j, be: (be[g], 0, j)),
        ],
        out_specs=pl.BlockSpec((BM, bn), lambda g, j, be: (g, j)),
    )
    return pl.pallas_call(
        functools.partial(_grouped_kernel, act),
        grid_spec=grid_spec,
        out_shape=jax.ShapeDtypeStruct((rows, n), jnp.float32),
    )(be, x, w, b2)


def _dense_kernel(act, x_ref, w_ref, b_ref, o_ref):
    y = lax.dot_general(
        x_ref[...], w_ref[...],
        (((1,), (1,)), ((), ())),
        preferred_element_type=jnp.float32,
    )
    y = y + b_ref[...]
    if act == "relu":
        y = jnp.maximum(y, 0.0)
    else:
        y = jnp.tanh(y)
    o_ref[...] = y


def _dense(x, w, b, act):
    rows = x.shape[0]
    n, k = w.shape
    return pl.pallas_call(
        functools.partial(_dense_kernel, act),
        grid=(rows // BM,),
        in_specs=[
            pl.BlockSpec((BM, k), lambda g: (g, 0)),
            pl.BlockSpec((n, k), lambda g: (0, 0)),
            pl.BlockSpec((1, n), lambda g: (0, 0)),
        ],
        out_specs=pl.BlockSpec((BM, n), lambda g: (g, 0)),
        out_shape=jax.ShapeDtypeStruct((rows, n), jnp.float32),
    )(x, w, b.reshape(1, n))


def kernel(in_list, o, i, W1, b1, W2, b2, W3, b3, W4, b4, W5, b5):
    x = jnp.concatenate([in_list[0], in_list[1]], axis=1)
    gidx_i, pos_i, be_i = _routing(i)
    gidx_o, pos_o, be_o = _routing(o)

    x_s = _gather_rows(x, gidx_i)
    h = _grouped_matmul(x_s, W1, b1, be_i, act="relu")
    h = _dense(h, W2, b2, "relu")
    h = _dense(h, W3, b3, "relu")
    h = _dense(h, W4, b4, "tanh")
    h = _gather_rows(h, pos_i[gidx_o])
    y = _grouped_matmul(h, W5, b5, be_o, act=None, nsplit=2)
    out = _gather_rows(y, pos_o)
    return tuple(jnp.split(out, 3, axis=1))


# speedup vs baseline: 1.6724x; 1.3012x over previous
import functools

import jax
import jax.numpy as jnp
from jax import lax
from jax.experimental import pallas as pl
from jax.experimental.pallas import tpu as pltpu

E = 8
IN_DIM = 1024
DIM = 2048
B = 4096
OUT_DIM = 3072

BM = 256
G = B // BM + E
P = G * BM


def _routing(idx):
    idx = idx.astype(jnp.int32)
    order = jnp.argsort(idx, stable=True).astype(jnp.int32)
    counts = jnp.bincount(idx, length=E)
    blocks = (counts + BM - 1) // BM
    ends_blk = jnp.cumsum(blocks)
    start_al = ((ends_blk - blocks) * BM).astype(jnp.int32)
    cum = (jnp.cumsum(counts) - counts).astype(jnp.int32)
    es = idx[order]
    rank = jnp.arange(B, dtype=jnp.int32) - cum[es]
    dst = start_al[es] + rank
    gidx = jnp.zeros((P,), jnp.int32).at[dst].set(order)
    pos = jnp.zeros((B,), jnp.int32).at[order].set(dst)
    be = jnp.searchsorted(ends_blk, jnp.arange(G), side="right")
    be = jnp.minimum(be, E - 1).astype(jnp.int32)
    return gidx, pos, be


def _gather_rows(table, idx):
    return jnp.take(table, idx, axis=0)


def _grouped_kernel(act, out_dtype, be_ref, x_ref, w_ref, b_ref, o_ref):
    y = lax.dot_general(
        x_ref[...], w_ref[0].astype(jnp.bfloat16),
        (((1,), (1,)), ((), ())),
        preferred_element_type=jnp.float32,
    )
    y = y + b_ref[0]
    if act == "relu":
        y = jnp.maximum(y, 0.0)
    o_ref[...] = y.astype(out_dtype)


def _grouped_matmul(x, w, b, be, act, nsplit=1, out_dtype=jnp.bfloat16):
    rows = x.shape[0]
    _, n, k = w.shape
    bn = n // nsplit
    b2 = b.reshape(E, 1, n)
    grid_spec = pltpu.PrefetchScalarGridSpec(
        num_scalar_prefetch=1,
        grid=(rows // BM, nsplit),
        in_specs=[
            pl.BlockSpec((BM, k), lambda g, j, be: (g, 0)),
            pl.BlockSpec((1, bn, k), lambda g, j, be: (be[g], j, 0)),
            pl.BlockSpec((1, 1, bn), lambda g, j, be: (be[g], 0, j)),
        ],
        out_specs=pl.BlockSpec((BM, bn), lambda g, j, be: (g, j)),
    )
    return pl.pallas_call(
        functools.partial(_grouped_kernel, act, out_dtype),
        grid_spec=grid_spec,
        out_shape=jax.ShapeDtypeStruct((rows, n), out_dtype),
    )(be, x, w, b2)


def _dense_kernel(act, x_ref, w_ref, b_ref, o_ref):
    y = lax.dot_general(
        x_ref[...], w_ref[...].astype(jnp.bfloat16),
        (((1,), (1,)), ((), ())),
        preferred_element_type=jnp.float32,
    )
    y = y + b_ref[...]
    if act == "relu":
        y = jnp.maximum(y, 0.0)
    else:
        y = jnp.tanh(y)
    o_ref[...] = y.astype(jnp.bfloat16)


def _dense(x, w, b, act):
    rows = x.shape[0]
    n, k = w.shape
    return pl.pallas_call(
        functools.partial(_dense_kernel, act),
        grid=(rows // BM,),
        in_specs=[
            pl.BlockSpec((BM, k), lambda g: (g, 0)),
            pl.BlockSpec((n, k), lambda g: (0, 0)),
            pl.BlockSpec((1, n), lambda g: (0, 0)),
        ],
        out_specs=pl.BlockSpec((BM, n), lambda g: (g, 0)),
        out_shape=jax.ShapeDtypeStruct((rows, n), jnp.bfloat16),
    )(x, w, b.reshape(1, n))


def kernel(in_list, o, i, W1, b1, W2, b2, W3, b3, W4, b4, W5, b5):
    x = jnp.concatenate([in_list[0], in_list[1]], axis=1)
    gidx_i, pos_i, be_i = _routing(i)
    gidx_o, pos_o, be_o = _routing(o)

    x_s = _gather_rows(x.astype(jnp.bfloat16), gidx_i)
    h = _grouped_matmul(x_s, W1, b1, be_i, act="relu")
    h = _dense(h, W2, b2, "relu")
    h = _dense(h, W3, b3, "relu")
    h = _dense(h, W4, b4, "tanh")
    h = _gather_rows(h, pos_i[gidx_o])
    y = _grouped_matmul(h, W5, b5, be_o, act=None, nsplit=2,
                        out_dtype=jnp.float32)
    out = _gather_rows(y, pos_o)
    return tuple(jnp.split(out, 3, axis=1))


# speedup vs baseline: 1.7339x; 1.0368x over previous
import functools

import jax
import jax.numpy as jnp
from jax import lax
from jax.experimental import pallas as pl
from jax.experimental.pallas import tpu as pltpu

E = 8
IN_DIM = 1024
DIM = 2048
B = 4096
OUT_DIM = 3072

BM = 256
G = B // BM + E
P = G * BM


def _routing(idx):
    idx = idx.astype(jnp.int32)
    order = jnp.argsort(idx, stable=True).astype(jnp.int32)
    counts = jnp.bincount(idx, length=E)
    blocks = (counts + BM - 1) // BM
    ends_blk = jnp.cumsum(blocks)
    start_al = ((ends_blk - blocks) * BM).astype(jnp.int32)
    cum = (jnp.cumsum(counts) - counts).astype(jnp.int32)
    es = idx[order]
    rank = jnp.arange(B, dtype=jnp.int32) - cum[es]
    dst = start_al[es] + rank
    gidx = jnp.zeros((P,), jnp.int32).at[dst].set(order)
    pos = jnp.zeros((B,), jnp.int32).at[order].set(dst)
    be = jnp.searchsorted(ends_blk, jnp.arange(G), side="right")
    be = jnp.minimum(be, E - 1).astype(jnp.int32)
    return gidx, pos, be


def _gather_rows(table, idx):
    return jnp.take(table, idx, axis=0)


def _grouped_kernel(act, out_dtype, be_ref, x_ref, w_ref, b_ref, o_ref, wb_ref):
    g = pl.program_id(1)
    prev = be_ref[jnp.maximum(g - 1, 0)]

    @pl.when(jnp.logical_or(g == 0, be_ref[g] != prev))
    def _cast():
        wb_ref[...] = w_ref[0].astype(jnp.bfloat16)

    y = lax.dot_general(
        x_ref[...], wb_ref[...],
        (((1,), (1,)), ((), ())),
        preferred_element_type=jnp.float32,
    )
    y = y + b_ref[0]
    if act == "relu":
        y = jnp.maximum(y, 0.0)
    o_ref[...] = y.astype(out_dtype)


def _grouped_matmul(x, w, b, be, act, nsplit=1, out_dtype=jnp.bfloat16):
    rows = x.shape[0]
    _, n, k = w.shape
    bn = n // nsplit
    b2 = b.reshape(E, 1, n)
    grid_spec = pltpu.PrefetchScalarGridSpec(
        num_scalar_prefetch=1,
        grid=(nsplit, rows // BM),
        in_specs=[
            pl.BlockSpec((BM, k), lambda j, g, be: (g, 0)),
            pl.BlockSpec((1, bn, k), lambda j, g, be: (be[g], j, 0)),
            pl.BlockSpec((1, 1, bn), lambda j, g, be: (be[g], 0, j)),
        ],
        out_specs=pl.BlockSpec((BM, bn), lambda j, g, be: (g, j)),
        scratch_shapes=[pltpu.VMEM((bn, k), jnp.bfloat16)],
    )
    return pl.pallas_call(
        functools.partial(_grouped_kernel, act, out_dtype),
        grid_spec=grid_spec,
        out_shape=jax.ShapeDtypeStruct((rows, n), out_dtype),
    )(be, x, w, b2)


def _dense_kernel(act, x_ref, w_ref, b_ref, o_ref, wb_ref):
    @pl.when(pl.program_id(0) == 0)
    def _cast():
        wb_ref[...] = w_ref[...].astype(jnp.bfloat16)

    y = lax.dot_general(
        x_ref[...], wb_ref[...],
        (((1,), (1,)), ((), ())),
        preferred_element_type=jnp.float32,
    )
    y = y + b_ref[...]
    if act == "relu":
        y = jnp.maximum(y, 0.0)
    else:
        y = jnp.tanh(y)
    o_ref[...] = y.astype(jnp.bfloat16)


def _dense(x, w, b, act):
    rows = x.shape[0]
    n, k = w.shape
    return pl.pallas_call(
        functools.partial(_dense_kernel, act),
        grid=(rows // BM,),
        in_specs=[
            pl.BlockSpec((BM, k), lambda g: (g, 0)),
            pl.BlockSpec((n, k), lambda g: (0, 0)),
            pl.BlockSpec((1, n), lambda g: (0, 0)),
        ],
        out_specs=pl.BlockSpec((BM, n), lambda g: (g, 0)),
        out_shape=jax.ShapeDtypeStruct((rows, n), jnp.bfloat16),
        scratch_shapes=[pltpu.VMEM((n, k), jnp.bfloat16)],
    )(x, w, b.reshape(1, n))


def kernel(in_list, o, i, W1, b1, W2, b2, W3, b3, W4, b4, W5, b5):
    x = jnp.concatenate([in_list[0], in_list[1]], axis=1)
    gidx_i, pos_i, be_i = _routing(i)
    gidx_o, pos_o, be_o = _routing(o)

    x_s = _gather_rows(x.astype(jnp.bfloat16), gidx_i)
    h = _grouped_matmul(x_s, W1, b1, be_i, act="relu")
    h = _dense(h, W2, b2, "relu")
    h = _dense(h, W3, b3, "relu")
    h = _dense(h, W4, b4, "tanh")
    h = _gather_rows(h, pos_i[gidx_o])
    y = _grouped_matmul(h, W5, b5, be_o, act=None, nsplit=2,
                        out_dtype=jnp.float32)
    out = _gather_rows(y, pos_o)
    return tuple(jnp.split(out, 3, axis=1))


# speedup vs baseline: 1.8643x; 1.0752x over previous
import functools

import jax
import jax.numpy as jnp
from jax import lax
from jax.experimental import pallas as pl
from jax.experimental.pallas import tpu as pltpu

E = 8
IN_DIM = 1024
DIM = 2048
B = 4096
OUT_DIM = 3072

BM = 256
G = B // BM + E
P = G * BM


def _routing(idx):
    idx = idx.astype(jnp.int32)
    order = jnp.argsort(idx, stable=True).astype(jnp.int32)
    counts = jnp.bincount(idx, length=E)
    blocks = (counts + BM - 1) // BM
    ends_blk = jnp.cumsum(blocks)
    start_al = ((ends_blk - blocks) * BM).astype(jnp.int32)
    cum = (jnp.cumsum(counts) - counts).astype(jnp.int32)
    es = idx[order]
    adj = start_al - cum
    dst = jnp.arange(B, dtype=jnp.int32) + adj[es]
    gidx = jnp.zeros((P,), jnp.int32).at[dst].set(order)
    pos = jnp.zeros((B,), jnp.int32).at[order].set(dst)
    nv = jnp.sum(blocks).astype(jnp.int32)
    be = jnp.searchsorted(ends_blk, jnp.minimum(jnp.arange(G), nv - 1),
                          side="right")
    be = jnp.minimum(be, E - 1).astype(jnp.int32)
    return gidx, pos, jnp.concatenate([be, nv[None]])


def _gather_rows(table, idx):
    return jnp.take(table, idx, axis=0, mode="clip")


def _grouped_kernel(act, out_dtype, be_ref, x_ref, w_ref, b_ref, o_ref, wb_ref):
    g = pl.program_id(1)
    prev = be_ref[jnp.maximum(g - 1, 0)]

    @pl.when(jnp.logical_and(g < be_ref[G],
                             jnp.logical_or(g == 0, be_ref[g] != prev)))
    def _cast():
        wb_ref[...] = w_ref[0].astype(jnp.bfloat16)

    @pl.when(g < be_ref[G])
    def _compute():
        y = lax.dot_general(
            x_ref[...], wb_ref[...],
            (((1,), (1,)), ((), ())),
            preferred_element_type=jnp.float32,
        )
        y = y + b_ref[0]
        if act == "relu":
            y = jnp.maximum(y, 0.0)
        o_ref[...] = y.astype(out_dtype)


def _grouped_matmul(x, w, b, be, act, nsplit=1, out_dtype=jnp.bfloat16):
    rows = x.shape[0]
    _, n, k = w.shape
    bn = n // nsplit
    b2 = b.reshape(E, 1, n)
    grid_spec = pltpu.PrefetchScalarGridSpec(
        num_scalar_prefetch=1,
        grid=(nsplit, rows // BM),
        in_specs=[
            pl.BlockSpec((BM, k), lambda j, g, be: (g, 0)),
            pl.BlockSpec((1, bn, k), lambda j, g, be: (be[g], j, 0)),
            pl.BlockSpec((1, 1, bn), lambda j, g, be: (be[g], 0, j)),
        ],
        out_specs=pl.BlockSpec((BM, bn), lambda j, g, be: (g, j)),
        scratch_shapes=[pltpu.VMEM((bn, k), jnp.bfloat16)],
    )
    return pl.pallas_call(
        functools.partial(_grouped_kernel, act, out_dtype),
        grid_spec=grid_spec,
        out_shape=jax.ShapeDtypeStruct((rows, n), out_dtype),
    )(be, x, w, b2)


def _dense_kernel(act, nv_ref, x_ref, w_ref, b_ref, o_ref, wb_ref):
    g = pl.program_id(0)

    @pl.when(g == 0)
    def _cast():
        wb_ref[...] = w_ref[...].astype(jnp.bfloat16)

    @pl.when(g < nv_ref[0])
    def _compute():
        y = lax.dot_general(
            x_ref[...], wb_ref[...],
            (((1,), (1,)), ((), ())),
            preferred_element_type=jnp.float32,
        )
        y = y + b_ref[...]
        if act == "relu":
            y = jnp.maximum(y, 0.0)
        else:
            y = jnp.tanh(y)
        o_ref[...] = y.astype(jnp.bfloat16)


def _dense(x, w, b, nv, act):
    rows = x.shape[0]
    n, k = w.shape
    grid_spec = pltpu.PrefetchScalarGridSpec(
        num_scalar_prefetch=1,
        grid=(rows // BM,),
        in_specs=[
            pl.BlockSpec((BM, k), lambda g, nv: (g, 0)),
            pl.BlockSpec((n, k), lambda g, nv: (0, 0)),
            pl.BlockSpec((1, n), lambda g, nv: (0, 0)),
        ],
        out_specs=pl.BlockSpec((BM, n), lambda g, nv: (g, 0)),
        scratch_shapes=[pltpu.VMEM((n, k), jnp.bfloat16)],
    )
    return pl.pallas_call(
        functools.partial(_dense_kernel, act),
        grid_spec=grid_spec,
        out_shape=jax.ShapeDtypeStruct((rows, n), jnp.bfloat16),
    )(nv, x, w, b.reshape(1, n))


def kernel(in_list, o, i, W1, b1, W2, b2, W3, b3, W4, b4, W5, b5):
    x = jnp.concatenate([in_list[0], in_list[1]], axis=1)
    gidx_i, pos_i, be_i = _routing(i)
    gidx_o, pos_o, be_o = _routing(o)

    nv_i = be_i[G:]
    x_s = _gather_rows(x.astype(jnp.bfloat16), gidx_i)
    h = _grouped_matmul(x_s, W1, b1, be_i, act="relu")
    h = _dense(h, W2, b2, nv_i, "relu")
    h = _dense(h, W3, b3, nv_i, "relu")
    h = _dense(h, W4, b4, nv_i, "tanh")
    h = _gather_rows(h, pos_i[gidx_o])
    y = _grouped_matmul(h, W5, b5, be_o, act=None, nsplit=2,
                        out_dtype=jnp.float32)
    return tuple(jnp.take(y[:, k * IN_DIM:(k + 1) * IN_DIM], pos_o,
                          axis=0, mode="clip")
                 for k in range(3))
